# baseline (speedup 1.0000x reference)
import jax
import jax.numpy as jnp
from jax.experimental import pallas as pl
from jax.experimental.pallas import tpu as pltpu
from jax.experimental.pallas import tpu_sc as plsc

_TT = 256
_GATHER_W = 128


def _code_body(z_ref, c2_ref, cb_ref, code_ref):
    z = z_ref[...]
    cb = cb_ref[...]
    dots_n2 = jax.lax.dot_general(
        z.astype(jnp.bfloat16), cb,
        dimension_numbers=(((1,), (1,)), ((), ())),
        preferred_element_type=jnp.float32,
    )
    z2 = jnp.sum(z * z, axis=1, keepdims=True)
    dist = (z2 + dots_n2) + c2_ref[...]
    idx = jnp.argmin(dist, axis=1)
    code_ref[...] = idx.astype(jnp.int32).reshape(code_ref.shape)


def _codes_tc(zf, cb_bf16, c2, tile0, nt):
    d = zf.shape[1]
    k = cb_bf16.shape[0]
    out = pl.pallas_call(
        _code_body,
        grid=(nt,),
        in_specs=[
            pl.BlockSpec((_TT, d), lambda i: (i + tile0, 0)),
            pl.BlockSpec((1, k), lambda i: (0, 0)),
            pl.BlockSpec((k, d), lambda i: (0, 0)),
        ],
        out_specs=pl.BlockSpec((1, 1, _TT), lambda i: (i, 0, 0)),
        out_shape=jax.ShapeDtypeStruct((nt, 1, _TT), jnp.int32),
        compiler_params=pltpu.CompilerParams(
            dimension_semantics=("parallel",),
        ),
    )(zf, c2, cb_bf16)
    return out.reshape(nt * _TT)


def _dequant_sc(codebook, codes_flat):
    n = codes_flat.shape[0]
    d = codebook.shape[1]
    idx2 = codes_flat.reshape(1, n)
    mesh = plsc.VectorSubcoreMesh(core_axis_name="c", subcore_axis_name="s")

    @pl.kernel(out_type=jax.ShapeDtypeStruct((n, d), codebook.dtype),
               mesh=mesh)
    def gather_kernel(cb_hbm, i_hbm, o_hbm):
        def body(i_vmem, o_vmem):
            pltpu.sync_copy(cb_hbm.at[i_vmem.at[0]], o_vmem)

        pltpu.emit_pipeline(
            body,
            grid=(n // _GATHER_W,),
            in_specs=[pl.BlockSpec((1, _GATHER_W), index_map=lambda i: (0, i))],
            out_specs=[pl.BlockSpec((_GATHER_W, d), index_map=lambda i: (i, 0))],
            core_axis_name=("c", "s"),
            dimension_semantics=(pltpu.PARALLEL,),
        )(i_hbm, o_hbm)

    return gather_kernel(codebook, idx2)


_CHUNKS = 2


def kernel(z_e, codebook):
    b, t, d = z_e.shape
    n = b * t
    zf = z_e.reshape(n, d)
    c2 = jnp.sum(codebook * codebook, axis=-1)[None, :]
    cb_n2 = (codebook * (-2.0)).astype(jnp.bfloat16)
    nt_chunk = n // (_TT * _CHUNKS)
    code_parts = []
    zq_parts = []
    for c in range(_CHUNKS):
        codes_c = _codes_tc(zf, cb_n2, c2, c * nt_chunk, nt_chunk)
        code_parts.append(codes_c)
        zq_parts.append(_dequant_sc(codebook, codes_c))
    codes = jnp.concatenate(code_parts)
    z_q = jnp.concatenate(zq_parts)
    return z_q.reshape(b, t, d), codes.reshape(b, t)


# speedup vs baseline: 1.0010x; 1.0010x over previous
import jax
import jax.numpy as jnp
from jax.experimental import pallas as pl
from jax.experimental.pallas import tpu as pltpu
from jax.experimental.pallas import tpu_sc as plsc

_TT = 256
_GATHER_W = 128


def _code_body(z_ref, c2_ref, cb_ref, code_ref):
    z = z_ref[...]
    cb = cb_ref[...]
    dots_n2 = jax.lax.dot_general(
        z.astype(jnp.bfloat16), cb,
        dimension_numbers=(((1,), (1,)), ((), ())),
        preferred_element_type=jnp.float32,
    )
    z2 = jnp.sum(z * z, axis=1, keepdims=True)
    dist = (z2 + dots_n2) + c2_ref[...]
    idx = jnp.argmin(dist, axis=1)
    code_ref[...] = idx.astype(jnp.int32).reshape(code_ref.shape)


def _codes_tc(zf, cb_bf16, c2, tile0, nt):
    d = zf.shape[1]
    k = cb_bf16.shape[0]
    out = pl.pallas_call(
        _code_body,
        grid=(nt,),
        in_specs=[
            pl.BlockSpec((_TT, d), lambda i: (i + tile0, 0)),
            pl.BlockSpec((1, k), lambda i: (0, 0)),
            pl.BlockSpec((k, d), lambda i: (0, 0)),
        ],
        out_specs=pl.BlockSpec((1, 1, _TT), lambda i: (i, 0, 0)),
        out_shape=jax.ShapeDtypeStruct((nt, 1, _TT), jnp.int32),
        compiler_params=pltpu.CompilerParams(
            dimension_semantics=("parallel",),
        ),
    )(zf, c2, cb_bf16)
    return out.reshape(nt * _TT)


def _dequant_sc(codebook, codes_flat):
    n = codes_flat.shape[0]
    d = codebook.shape[1]
    idx2 = codes_flat.reshape(1, n)
    mesh = plsc.VectorSubcoreMesh(core_axis_name="c", subcore_axis_name="s")

    @pl.kernel(out_type=jax.ShapeDtypeStruct((n, d), codebook.dtype),
               mesh=mesh)
    def gather_kernel(cb_hbm, i_hbm, o_hbm):
        def body(i_vmem, o_vmem):
            pltpu.sync_copy(cb_hbm.at[i_vmem.at[0]], o_vmem)

        pltpu.emit_pipeline(
            body,
            grid=(n // _GATHER_W,),
            in_specs=[pl.BlockSpec((1, _GATHER_W), index_map=lambda i: (0, i))],
            out_specs=[pl.BlockSpec((_GATHER_W, d), index_map=lambda i: (i, 0))],
            core_axis_name=("c", "s"),
            dimension_semantics=(pltpu.PARALLEL,),
        )(i_hbm, o_hbm)

    return gather_kernel(codebook, idx2)


_CHUNKS = 2


def kernel(z_e, codebook):
    b, t, d = z_e.shape
    n = b * t
    zf = z_e.reshape(n, d)
    c2 = jnp.sum(codebook * codebook, axis=-1)[None, :]
    cb_n2 = (codebook * (-2.0)).astype(jnp.bfloat16)
    nt_chunk = n // (_TT * _CHUNKS)
    code_parts = [
        _codes_tc(zf, cb_n2, c2, c * nt_chunk, nt_chunk)
        for c in range(_CHUNKS)
    ]
    zq_parts = [_dequant_sc(codebook, codes_c) for codes_c in code_parts]
    codes = jnp.concatenate(code_parts)
    z_q = jnp.concatenate(zq_parts)
    return z_q.reshape(b, t, d), codes.reshape(b, t)


# speedup vs baseline: 1.0277x; 1.0267x over previous
import jax
import jax.numpy as jnp
from jax.experimental import pallas as pl
from jax.experimental.pallas import tpu as pltpu
from jax.experimental.pallas import tpu_sc as plsc

_TT = 256
_GATHER_W = 128


def _code_body(z_ref, c2_ref, cb_ref, code_ref):
    z = z_ref[...]
    cb = cb_ref[...]
    dots_n2 = jax.lax.dot_general(
        z.astype(jnp.bfloat16), cb,
        dimension_numbers=(((1,), (1,)), ((), ())),
        preferred_element_type=jnp.float32,
    )
    z2 = jnp.sum(z * z, axis=1, keepdims=True)
    dist = (z2 + dots_n2) + c2_ref[...]
    idx = jnp.argmin(dist, axis=1)
    code_ref[...] = idx.astype(jnp.int32).reshape(code_ref.shape)


def _codes_tc(zf, cb_bf16, c2, tile0, nt):
    d = zf.shape[1]
    k = cb_bf16.shape[0]
    out = pl.pallas_call(
        _code_body,
        grid=(nt,),
        in_specs=[
            pl.BlockSpec((_TT, d), lambda i: (i + tile0, 0)),
            pl.BlockSpec((1, k), lambda i: (0, 0)),
            pl.BlockSpec((k, d), lambda i: (0, 0)),
        ],
        out_specs=pl.BlockSpec((1, 1, _TT), lambda i: (i, 0, 0)),
        out_shape=jax.ShapeDtypeStruct((nt, 1, _TT), jnp.int32),
        compiler_params=pltpu.CompilerParams(
            dimension_semantics=("parallel",),
        ),
    )(zf, c2, cb_bf16)
    return out.reshape(nt * _TT)


def _dequant_sc(codebook, codes_flat):
    n = codes_flat.shape[0]
    d = codebook.shape[1]
    idx2 = codes_flat.reshape(1, n)
    mesh = plsc.VectorSubcoreMesh(core_axis_name="c", subcore_axis_name="s")

    @pl.kernel(out_type=jax.ShapeDtypeStruct((n, d), codebook.dtype),
               mesh=mesh)
    def gather_kernel(cb_hbm, i_hbm, o_hbm):
        def body(i_vmem, o_vmem):
            pltpu.sync_copy(cb_hbm.at[i_vmem.at[0]], o_vmem)

        pltpu.emit_pipeline(
            body,
            grid=(n // _GATHER_W,),
            in_specs=[pl.BlockSpec((1, _GATHER_W), index_map=lambda i: (0, i))],
            out_specs=[pl.BlockSpec((_GATHER_W, d), index_map=lambda i: (i, 0))],
            core_axis_name=("c", "s"),
            dimension_semantics=(pltpu.PARALLEL,),
        )(i_hbm, o_hbm)

    return gather_kernel(codebook, idx2)


def kernel(z_e, codebook):
    b, t, d = z_e.shape
    n = b * t
    zf = z_e.reshape(n, d)
    c2 = jnp.sum(codebook * codebook, axis=-1)[None, :]
    cb_n2 = (codebook * (-2.0)).astype(jnp.bfloat16)
    codes = _codes_tc(zf, cb_n2, c2, 0, n // _TT)
    z_q = _dequant_sc(codebook, codes)
    return z_q.reshape(b, t, d), codes.reshape(b, t)
